# baseline (speedup 1.0000x reference)
import functools

import jax
import jax.numpy as jnp
from jax import lax
from jax.experimental import pallas as pl
from jax.experimental.pallas import tpu as pltpu
from jax.experimental.pallas import tpu_sc as plsc

EMBED = 128
BATCH = 16384
NFEAT = 4
NC, NS, LANES = 2, 16, 16
NW = NC * NS
BPW = BATCH // NW
RPW = BPW * NFEAT
CH = 128
NCH = RPW // CH
ROWS = 47


def _body(x_hbm, table_hbm, out_hbm, xv, idx_v, rows0, rows1, sem0, sem1):
    wid = lax.axis_index("s") * NC + lax.axis_index("c")
    base_r = wid * RPW

    pltpu.sync_copy(x_hbm.at[pl.ds(base_r, RPW)], xv)

    lane = lax.iota(jnp.int32, 16)
    feat = lane % NFEAT
    off = jnp.where(feat == 0, 0,
                    jnp.where(feat == 1, 4,
                              jnp.where(feat == 2, 16, 23)))

    for c in range(NCH):
        for k in range(CH // 16):
            g = xv[pl.ds((c * 8 + k) * 16, 16)]
            idx_v[c, pl.ds(k * 16, 16)] = g + off

    bufs = (rows0, rows1)
    sems = (sem0, sem1)
    cps = [None, None]
    cps[0] = pltpu.async_copy(table_hbm.at[idx_v.at[0]], bufs[0], sems[0])
    for c in range(NCH):
        n = c + 1
        if n < NCH:
            cps[n % 2] = pltpu.async_copy(
                table_hbm.at[idx_v.at[n]], bufs[n % 2], sems[n % 2])
        cps[c % 2].wait()
        pltpu.sync_copy(bufs[c % 2], out_hbm.at[pl.ds(base_r + c * CH, CH)])


_gather = functools.partial(
    pl.kernel,
    out_type=jax.ShapeDtypeStruct((BATCH * NFEAT, EMBED), jnp.float32),
    mesh=plsc.VectorSubcoreMesh(core_axis_name="c", subcore_axis_name="s"),
    scratch_types=[
        pltpu.VMEM((RPW,), jnp.int32),
        pltpu.VMEM((NCH, CH), jnp.int32),
        pltpu.VMEM((CH, EMBED), jnp.float32),
        pltpu.VMEM((CH, EMBED), jnp.float32),
        pltpu.SemaphoreType.DMA,
        pltpu.SemaphoreType.DMA,
    ],
)(_body)


@jax.jit
def kernel(x, W_season, W_month, W_day_of_week, W_hour):
    xi = x[:, 2:6].astype(jnp.int32).reshape(BATCH * NFEAT)
    table = jnp.concatenate([W_season, W_month, W_day_of_week, W_hour], axis=0)
    out = _gather(xi, table)
    return out.reshape(BATCH, NFEAT * EMBED)


# speedup vs baseline: 4.0809x; 4.0809x over previous
import functools

import jax
import jax.numpy as jnp
from jax import lax
from jax.experimental import pallas as pl
from jax.experimental.pallas import tpu as pltpu
from jax.experimental.pallas import tpu_sc as plsc

EMBED = 128
BATCH = 16384
NFEAT = 4
NC, NS, LANES = 2, 16, 16
NW = NC * NS
BPW = BATCH // NW
RPW = BPW * NFEAT
CH = 128
NCH = RPW // CH
ROWS = 47


def _body(x_hbm, table_hbm, out_hbm, xv, idx_v, table_sp, rows0, rows1,
          sem0, sem1):
    sid = lax.axis_index("s")
    wid = sid * NC + lax.axis_index("c")
    base_r = wid * RPW

    @pl.when(sid == 0)
    def _():
        pltpu.sync_copy(table_hbm, table_sp)

    pltpu.sync_copy(x_hbm.at[pl.ds(base_r, RPW)], xv)

    lane = lax.iota(jnp.int32, 16)
    feat = lane % NFEAT
    off = jnp.where(feat == 0, 0,
                    jnp.where(feat == 1, 4,
                              jnp.where(feat == 2, 16, 23)))

    for c in range(NCH):
        for k in range(CH // 16):
            g = xv[pl.ds((c * 8 + k) * 16, 16)]
            idx_v[c, pl.ds(k * 16, 16)] = g + off

    plsc.subcore_barrier()

    bufs = (rows0, rows1)
    sems = (sem0, sem1)
    cps = [None, None]
    cps[0] = pltpu.async_copy(table_sp.at[idx_v.at[0]], bufs[0], sems[0])
    for c in range(NCH):
        n = c + 1
        if n < NCH:
            cps[n % 2] = pltpu.async_copy(
                table_sp.at[idx_v.at[n]], bufs[n % 2], sems[n % 2])
        cps[c % 2].wait()
        pltpu.sync_copy(bufs[c % 2], out_hbm.at[pl.ds(base_r + c * CH, CH)])


_gather = functools.partial(
    pl.kernel,
    out_type=jax.ShapeDtypeStruct((BATCH * NFEAT, EMBED), jnp.float32),
    mesh=plsc.VectorSubcoreMesh(core_axis_name="c", subcore_axis_name="s"),
    scratch_types=[
        pltpu.VMEM((RPW,), jnp.int32),
        pltpu.VMEM((NCH, CH), jnp.int32),
        pltpu.VMEM_SHARED((ROWS, EMBED), jnp.float32),
        pltpu.VMEM((CH, EMBED), jnp.float32),
        pltpu.VMEM((CH, EMBED), jnp.float32),
        pltpu.SemaphoreType.DMA,
        pltpu.SemaphoreType.DMA,
    ],
)(_body)


@jax.jit
def kernel(x, W_season, W_month, W_day_of_week, W_hour):
    xi = x[:, 2:6].astype(jnp.int32).reshape(BATCH * NFEAT)
    table = jnp.concatenate([W_season, W_month, W_day_of_week, W_hour], axis=0)
    out = _gather(xi, table)
    return out.reshape(BATCH, NFEAT * EMBED)


# speedup vs baseline: 6.4088x; 1.5704x over previous
import functools

import jax
import jax.numpy as jnp
from jax import lax
from jax.experimental import pallas as pl
from jax.experimental.pallas import tpu as pltpu
from jax.experimental.pallas import tpu_sc as plsc

EMBED = 128
BATCH = 16384
NFEAT = 4
NC, NS = 2, 16
NW = NC * NS
BPW = BATCH // NW
CH = 64
NBLK = BPW // CH
TABLE_ROWS = (4, 12, 7, 24)


def _body(x_hbm, t0, t1, t2, t3, out_hbm,
          colbuf, ts0, ts1, ts2, ts3, rows0, rows1, sem0, sem1):
    sid = lax.axis_index("s")
    wid = sid * NC + lax.axis_index("c")
    base_b = wid * BPW

    tables_sp = (ts0, ts1, ts2, ts3)

    @pl.when(sid == 0)
    def _():
        for th, tsp in zip((t0, t1, t2, t3), tables_sp):
            pltpu.sync_copy(th, tsp)

    for f in range(NFEAT):
        for blk in range(NBLK):
            pltpu.sync_copy(
                x_hbm.at[pl.ds(f * BATCH + base_b + blk * CH, CH)],
                colbuf.at[f * NBLK + blk])

    plsc.subcore_barrier()

    bufs = (rows0, rows1)
    sems = (sem0, sem1)

    def start(i):
        blk, f = divmod(i, NFEAT)
        return pltpu.async_copy(
            tables_sp[f].at[colbuf.at[f * NBLK + blk]],
            bufs[i % 2], sems[i % 2])

    n_tasks = NBLK * NFEAT
    cps = [start(0), None]
    for i in range(n_tasks):
        if i + 1 < n_tasks:
            cps[(i + 1) % 2] = start(i + 1)
        blk, f = divmod(i, NFEAT)
        cps[i % 2].wait()
        pltpu.sync_copy(
            bufs[i % 2],
            out_hbm.at[pl.ds(base_b + blk * CH, CH),
                       pl.ds(f * EMBED, EMBED)])


_gather = functools.partial(
    pl.kernel,
    out_type=jax.ShapeDtypeStruct((BATCH, NFEAT * EMBED), jnp.float32),
    mesh=plsc.VectorSubcoreMesh(core_axis_name="c", subcore_axis_name="s"),
    scratch_types=[
        pltpu.VMEM((NFEAT * NBLK, CH), jnp.int32),
        pltpu.VMEM_SHARED((TABLE_ROWS[0], EMBED), jnp.float32),
        pltpu.VMEM_SHARED((TABLE_ROWS[1], EMBED), jnp.float32),
        pltpu.VMEM_SHARED((TABLE_ROWS[2], EMBED), jnp.float32),
        pltpu.VMEM_SHARED((TABLE_ROWS[3], EMBED), jnp.float32),
        pltpu.VMEM((CH, EMBED), jnp.float32),
        pltpu.VMEM((CH, EMBED), jnp.float32),
        pltpu.SemaphoreType.DMA,
        pltpu.SemaphoreType.DMA,
    ],
)(_body)


@jax.jit
def kernel(x, W_season, W_month, W_day_of_week, W_hour):
    xt = x[:, 2:6].astype(jnp.int32).T.reshape(NFEAT * BATCH)
    return _gather(xt, W_season, W_month, W_day_of_week, W_hour)
